# baseline (speedup 1.0000x reference)
import functools

import jax
import jax.numpy as jnp
from jax import lax
from jax.experimental import pallas as pl
from jax.experimental.pallas import tpu as pltpu
from jax.experimental.pallas import tpu_sc as plsc
from jax.experimental.layout import Format, Layout

TOP_K = 2
N_EXPERTS = 16
D_MODEL = 2048
N_TOKENS = 16384

NUM_CORES = 2
NUM_SUBCORES = 16
NUM_WORKERS = NUM_CORES * NUM_SUBCORES
TOK_PER_WORKER = N_TOKENS // NUM_WORKERS
LANES = 16
BLOCKS_PER_WORKER = TOK_PER_WORKER // LANES

_NEG_BIG = -3.0e38


def _proj_body(*refs):
    n_split = (len(refs) - 2) // 2
    x_refs = refs[:n_split]
    wt_refs = refs[n_split:2 * n_split]
    b_ref = refs[2 * n_split]
    out_ref = refs[2 * n_split + 1]
    acc = b_ref[...].astype(jnp.float32)
    for x_ref, wt_ref in zip(x_refs, wt_refs):
        acc = acc + jnp.dot(
            x_ref[...], wt_ref[...], preferred_element_type=jnp.float32
        )
    out_ref[...] = acc


def _project(xf, wt, b2d, block_m, n_split=1):
    grid = (N_TOKENS // block_m,)
    kd = D_MODEL // n_split
    x_specs = [
        pl.BlockSpec((block_m, kd), lambda i, j=j: (i, j))
        for j in range(n_split)
    ]
    wt_specs = [
        pl.BlockSpec((kd, N_EXPERTS), lambda i, j=j: (j, 0))
        for j in range(n_split)
    ]
    return pl.pallas_call(
        _proj_body,
        grid=grid,
        in_specs=x_specs + wt_specs + [pl.BlockSpec((1, N_EXPERTS), lambda i: (0, 0))],
        out_specs=pl.BlockSpec((block_m, N_EXPERTS), lambda i: (i, 0)),
        out_shape=jax.ShapeDtypeStruct((N_TOKENS, N_EXPERTS), jnp.float32),
    )(*([xf] * n_split + [wt] * n_split + [b2d]))


def _router_body(logits_hbm, tkw_hbm, tki_hbm, rw_hbm, lg_v, tkw_v, tki_v, rw_v):
    wid = lax.axis_index("s") * NUM_CORES + lax.axis_index("c")
    base = wid * TOK_PER_WORKER

    pltpu.sync_copy(logits_hbm.at[pl.ds(base, TOK_PER_WORKER)], lg_v)

    lane = lax.broadcasted_iota(jnp.int32, (LANES,), 0)

    def block(t, carry):
        toks = t * LANES + lane
        cols = [
            plsc.load_gather(lg_v, [toks, jnp.full((LANES,), e, jnp.int32)])
            for e in range(N_EXPERTS)
        ]
        m1 = cols[0]
        i1 = jnp.zeros((LANES,), jnp.int32)
        for e in range(1, N_EXPERTS):
            gt = cols[e] > m1
            m1 = jnp.where(gt, cols[e], m1)
            i1 = jnp.where(gt, jnp.int32(e), i1)
        m2 = jnp.full((LANES,), _NEG_BIG, jnp.float32)
        i2 = jnp.zeros((LANES,), jnp.int32)
        for e in range(N_EXPERTS):
            cand = jnp.where(i1 == e, _NEG_BIG, cols[e])
            gt = cand > m2
            m2 = jnp.where(gt, cand, m2)
            i2 = jnp.where(gt, jnp.int32(e), i2)
        s1 = 1.0 / (1.0 + jnp.exp(-m1))
        s2 = 1.0 / (1.0 + jnp.exp(-m2))

        zero_i = jnp.zeros((LANES,), jnp.int32)
        one_i = jnp.full((LANES,), 1, jnp.int32)
        plsc.store_scatter(tkw_v, [toks, zero_i], s1)
        plsc.store_scatter(tkw_v, [toks, one_i], s2)
        plsc.store_scatter(tki_v, [toks, zero_i], i1)
        plsc.store_scatter(tki_v, [toks, one_i], i2)
        zf = jnp.zeros((LANES,), jnp.float32)
        for e in range(N_EXPERTS):
            col = jnp.where(i1 == e, s1, jnp.where(i2 == e, s2, zf))
            plsc.store_scatter(rw_v, [toks, jnp.full((LANES,), e, jnp.int32)], col)
        return carry

    lax.fori_loop(0, BLOCKS_PER_WORKER, block, jnp.int32(0))

    pltpu.sync_copy(tkw_v, tkw_hbm.at[pl.ds(base, TOK_PER_WORKER)])
    pltpu.sync_copy(tki_v, tki_hbm.at[pl.ds(base, TOK_PER_WORKER)])
    pltpu.sync_copy(rw_v, rw_hbm.at[pl.ds(base, TOK_PER_WORKER)])


_route = functools.partial(
    pl.kernel,
    out_type=[
        jax.ShapeDtypeStruct((N_TOKENS, TOP_K), jnp.float32),
        jax.ShapeDtypeStruct((N_TOKENS, TOP_K), jnp.int32),
        jax.ShapeDtypeStruct((N_TOKENS, N_EXPERTS), jnp.float32),
    ],
    mesh=plsc.VectorSubcoreMesh(core_axis_name="c", subcore_axis_name="s"),
    scratch_types=[
        pltpu.VMEM((TOK_PER_WORKER, N_EXPERTS), jnp.float32),
        pltpu.VMEM((TOK_PER_WORKER, TOP_K), jnp.float32),
        pltpu.VMEM((TOK_PER_WORKER, TOP_K), jnp.int32),
        pltpu.VMEM((TOK_PER_WORKER, N_EXPERTS), jnp.float32),
    ],
    compiler_params=pltpu.CompilerParams(
        needs_layout_passes=False, use_tc_tiling_on_sc=False
    ),
)(_router_body)


def _kernel_impl(x, W, b):
    xf = x.reshape(N_TOKENS, D_MODEL)
    wt = W.T
    b2d = b.reshape(1, N_EXPERTS)
    logits = _project(xf, wt, b2d, block_m=1024, n_split=1)
    top_k_weight, top_k_idx, router_weight = _route(logits)
    return top_k_weight, top_k_idx, router_weight


@functools.lru_cache(maxsize=None)
def _jitted_kernel():
    dev = jax.devices()[0]
    fmt = Format(
        Layout(major_to_minor=(0, 1), tiling=()),
        jax.sharding.SingleDeviceSharding(dev),
    )
    return jax.jit(_kernel_impl, out_shardings=(fmt, fmt, fmt))


def kernel(x, W, b):
    return _jitted_kernel()(x, W, b)


# speedup vs baseline: 1.0539x; 1.0539x over previous
import functools

import jax
import jax.numpy as jnp
from jax import lax
from jax.experimental import pallas as pl
from jax.experimental.pallas import tpu as pltpu
from jax.experimental.pallas import tpu_sc as plsc

TOP_K = 2
N_EXPERTS = 16
D_MODEL = 2048
N_TOKENS = 16384

NUM_CORES = 2
NUM_SUBCORES = 16
NUM_WORKERS = NUM_CORES * NUM_SUBCORES
TOK_PER_WORKER = N_TOKENS // NUM_WORKERS
LANES = 16
BLOCKS_PER_WORKER = TOK_PER_WORKER // LANES

_NEG_BIG = -3.0e38


def _proj_body(x_ref, wt_ref, b_ref, out_ref):
    acc = jnp.dot(x_ref[...], wt_ref[...], preferred_element_type=jnp.float32)
    out_ref[...] = jnp.transpose(acc + b_ref[...])


def _project_t(xf, wt, b2d, block_m):
    grid = (N_TOKENS // block_m,)
    return pl.pallas_call(
        _proj_body,
        grid=grid,
        in_specs=[
            pl.BlockSpec((block_m, D_MODEL), lambda i: (i, 0)),
            pl.BlockSpec((D_MODEL, N_EXPERTS), lambda i: (0, 0)),
            pl.BlockSpec((1, N_EXPERTS), lambda i: (0, 0)),
        ],
        out_specs=pl.BlockSpec((N_EXPERTS, block_m), lambda i: (0, i)),
        out_shape=jax.ShapeDtypeStruct((N_EXPERTS, N_TOKENS), jnp.float32),
    )(xf, wt, b2d)


def _router_body(lgt_hbm, tkwt_hbm, tkit_hbm, rwt_hbm, lg_v, tkw_v, tki_v, rw_v):
    wid = lax.axis_index("s") * NUM_CORES + lax.axis_index("c")
    base = wid * TOK_PER_WORKER

    pltpu.sync_copy(lgt_hbm.at[:, pl.ds(base, TOK_PER_WORKER)], lg_v)

    def block(t, carry):
        toff = t * LANES
        cols = [lg_v[e, pl.ds(toff, LANES)] for e in range(N_EXPERTS)]
        m1 = cols[0]
        i1 = jnp.zeros((LANES,), jnp.int32)
        for e in range(1, N_EXPERTS):
            gt = cols[e] > m1
            m1 = jnp.where(gt, cols[e], m1)
            i1 = jnp.where(gt, jnp.int32(e), i1)
        m2 = jnp.full((LANES,), _NEG_BIG, jnp.float32)
        i2 = jnp.zeros((LANES,), jnp.int32)
        for e in range(N_EXPERTS):
            cand = jnp.where(i1 == e, _NEG_BIG, cols[e])
            gt = cand > m2
            m2 = jnp.where(gt, cand, m2)
            i2 = jnp.where(gt, jnp.int32(e), i2)
        s1 = 1.0 / (1.0 + jnp.exp(-m1))
        s2 = 1.0 / (1.0 + jnp.exp(-m2))

        tkw_v[0, pl.ds(toff, LANES)] = s1
        tkw_v[1, pl.ds(toff, LANES)] = s2
        tki_v[0, pl.ds(toff, LANES)] = i1
        tki_v[1, pl.ds(toff, LANES)] = i2
        zf = jnp.zeros((LANES,), jnp.float32)
        for e in range(N_EXPERTS):
            rw_v[e, pl.ds(toff, LANES)] = jnp.where(
                i1 == e, s1, jnp.where(i2 == e, s2, zf)
            )
        return carry

    lax.fori_loop(0, BLOCKS_PER_WORKER, block, jnp.int32(0))

    pltpu.sync_copy(tkw_v, tkwt_hbm.at[:, pl.ds(base, TOK_PER_WORKER)])
    pltpu.sync_copy(tki_v, tkit_hbm.at[:, pl.ds(base, TOK_PER_WORKER)])
    pltpu.sync_copy(rw_v, rwt_hbm.at[:, pl.ds(base, TOK_PER_WORKER)])


_route_t = functools.partial(
    pl.kernel,
    out_type=[
        jax.ShapeDtypeStruct((TOP_K, N_TOKENS), jnp.float32),
        jax.ShapeDtypeStruct((TOP_K, N_TOKENS), jnp.int32),
        jax.ShapeDtypeStruct((N_EXPERTS, N_TOKENS), jnp.float32),
    ],
    mesh=plsc.VectorSubcoreMesh(core_axis_name="c", subcore_axis_name="s"),
    scratch_types=[
        pltpu.VMEM((N_EXPERTS, TOK_PER_WORKER), jnp.float32),
        pltpu.VMEM((TOP_K, TOK_PER_WORKER), jnp.float32),
        pltpu.VMEM((TOP_K, TOK_PER_WORKER), jnp.int32),
        pltpu.VMEM((N_EXPERTS, TOK_PER_WORKER), jnp.float32),
    ],
    compiler_params=pltpu.CompilerParams(
        needs_layout_passes=False, use_tc_tiling_on_sc=False
    ),
)(_router_body)


def _epi_body(tkwt_ref, tkit_ref, rwt_ref, tkw_ref, tki_ref, rw_ref):
    tkw_ref[...] = jnp.transpose(tkwt_ref[...])
    tki_ref[...] = jnp.transpose(tkit_ref[...])
    rw_ref[...] = jnp.transpose(rwt_ref[...])


def _epilogue(tkwt, tkit, rwt, block_m):
    grid = (N_TOKENS // block_m,)
    return pl.pallas_call(
        _epi_body,
        grid=grid,
        in_specs=[
            pl.BlockSpec((TOP_K, block_m), lambda i: (0, i)),
            pl.BlockSpec((TOP_K, block_m), lambda i: (0, i)),
            pl.BlockSpec((N_EXPERTS, block_m), lambda i: (0, i)),
        ],
        out_specs=[
            pl.BlockSpec((block_m, TOP_K), lambda i: (i, 0)),
            pl.BlockSpec((block_m, TOP_K), lambda i: (i, 0)),
            pl.BlockSpec((block_m, N_EXPERTS), lambda i: (i, 0)),
        ],
        out_shape=[
            jax.ShapeDtypeStruct((N_TOKENS, TOP_K), jnp.float32),
            jax.ShapeDtypeStruct((N_TOKENS, TOP_K), jnp.int32),
            jax.ShapeDtypeStruct((N_TOKENS, N_EXPERTS), jnp.float32),
        ],
    )(tkwt, tkit, rwt)


@jax.jit
def kernel(x, W, b):
    xf = x.reshape(N_TOKENS, D_MODEL)
    wt = W.T
    b2d = b.reshape(1, N_EXPERTS)
    logits_t = _project_t(xf, wt, b2d, block_m=1024)
    tkwt, tkit, rwt = _route_t(logits_t)
    return _epilogue(tkwt, tkit, rwt, block_m=2048)


# speedup vs baseline: 1.4586x; 1.3840x over previous
import functools

import jax
import jax.numpy as jnp
from jax import lax
from jax.experimental import pallas as pl
from jax.experimental.pallas import tpu as pltpu
from jax.experimental.pallas import tpu_sc as plsc

TOP_K = 2
N_EXPERTS = 16
D_MODEL = 2048
N_TOKENS = 16384

NUM_CORES = 2
NUM_SUBCORES = 16
NUM_WORKERS = NUM_CORES * NUM_SUBCORES
TOK_PER_WORKER = N_TOKENS // NUM_WORKERS
LANES = 16
BLOCKS_PER_WORKER = TOK_PER_WORKER // LANES

_NEG_BIG = -3.0e38


def _proj_body(x_ref, wt_ref, b_ref, out_ref):
    acc = jnp.dot(x_ref[...], wt_ref[...], preferred_element_type=jnp.float32)
    out_ref[...] = jnp.transpose(acc + b_ref[...])


def _project_t(xf, wt, b2d, block_m):
    grid = (N_TOKENS // block_m,)
    return pl.pallas_call(
        _proj_body,
        grid=grid,
        in_specs=[
            pl.BlockSpec((block_m, D_MODEL), lambda i: (i, 0)),
            pl.BlockSpec((D_MODEL, N_EXPERTS), lambda i: (0, 0)),
            pl.BlockSpec((1, N_EXPERTS), lambda i: (0, 0)),
        ],
        out_specs=pl.BlockSpec((N_EXPERTS, block_m), lambda i: (0, i)),
        out_shape=jax.ShapeDtypeStruct((N_EXPERTS, N_TOKENS), jnp.float32),
    )(xf, wt, b2d)


def _router_body(lgt_hbm, tkwt_hbm, tkit_hbm, rwt_hbm, lg_v, tkw_v, tki_v, rw_v):
    wid = lax.axis_index("s") * NUM_CORES + lax.axis_index("c")
    base = wid * TOK_PER_WORKER

    pltpu.sync_copy(lgt_hbm.at[:, pl.ds(base, TOK_PER_WORKER)], lg_v)

    def block(t, carry):
        toff = t * LANES
        cols = [lg_v[e, pl.ds(toff, LANES)] for e in range(N_EXPERTS)]
        m1 = cols[0]
        i1 = jnp.zeros((LANES,), jnp.int32)
        for e in range(1, N_EXPERTS):
            gt = cols[e] > m1
            m1 = jnp.where(gt, cols[e], m1)
            i1 = jnp.where(gt, jnp.int32(e), i1)
        m2 = jnp.full((LANES,), _NEG_BIG, jnp.float32)
        i2 = jnp.zeros((LANES,), jnp.int32)
        for e in range(N_EXPERTS):
            cand = jnp.where(i1 == e, _NEG_BIG, cols[e])
            gt = cand > m2
            m2 = jnp.where(gt, cand, m2)
            i2 = jnp.where(gt, jnp.int32(e), i2)
        s1 = 1.0 / (1.0 + jnp.exp(-m1))
        s2 = 1.0 / (1.0 + jnp.exp(-m2))

        tkw_v[0, pl.ds(toff, LANES)] = s1
        tkw_v[1, pl.ds(toff, LANES)] = s2
        tki_v[0, pl.ds(toff, LANES)] = i1
        tki_v[1, pl.ds(toff, LANES)] = i2
        zf = jnp.zeros((LANES,), jnp.float32)
        for e in range(N_EXPERTS):
            rw_v[e, pl.ds(toff, LANES)] = jnp.where(
                i1 == e, s1, jnp.where(i2 == e, s2, zf)
            )
        return carry

    lax.fori_loop(0, BLOCKS_PER_WORKER, block, jnp.int32(0))

    pltpu.sync_copy(tkw_v, tkwt_hbm.at[:, pl.ds(base, TOK_PER_WORKER)])
    pltpu.sync_copy(tki_v, tkit_hbm.at[:, pl.ds(base, TOK_PER_WORKER)])
    pltpu.sync_copy(rw_v, rwt_hbm.at[:, pl.ds(base, TOK_PER_WORKER)])


_route_t = functools.partial(
    pl.kernel,
    out_type=[
        jax.ShapeDtypeStruct((TOP_K, N_TOKENS), jnp.float32),
        jax.ShapeDtypeStruct((TOP_K, N_TOKENS), jnp.int32),
        jax.ShapeDtypeStruct((N_EXPERTS, N_TOKENS), jnp.float32),
    ],
    mesh=plsc.VectorSubcoreMesh(core_axis_name="c", subcore_axis_name="s"),
    scratch_types=[
        pltpu.VMEM((N_EXPERTS, TOK_PER_WORKER), jnp.float32),
        pltpu.VMEM((TOP_K, TOK_PER_WORKER), jnp.float32),
        pltpu.VMEM((TOP_K, TOK_PER_WORKER), jnp.int32),
        pltpu.VMEM((N_EXPERTS, TOK_PER_WORKER), jnp.float32),
    ],
    compiler_params=pltpu.CompilerParams(
        needs_layout_passes=False, use_tc_tiling_on_sc=False
    ),
)(_router_body)


def _epi_body(tkwt_ref, tkit_ref, rwt_ref, tkw_ref, tki_ref, rw_ref):
    tkw_ref[...] = jnp.transpose(tkwt_ref[...])
    tki_ref[...] = jnp.transpose(tkit_ref[...])
    rw_ref[...] = jnp.transpose(rwt_ref[...])


def _epilogue(tkwt, tkit, rwt, block_m):
    grid = (N_TOKENS // block_m,)
    return pl.pallas_call(
        _epi_body,
        grid=grid,
        in_specs=[
            pl.BlockSpec((TOP_K, block_m), lambda i: (0, i)),
            pl.BlockSpec((TOP_K, block_m), lambda i: (0, i)),
            pl.BlockSpec((N_EXPERTS, block_m), lambda i: (0, i)),
        ],
        out_specs=[
            pl.BlockSpec((block_m, TOP_K), lambda i: (i, 0)),
            pl.BlockSpec((block_m, TOP_K), lambda i: (i, 0)),
            pl.BlockSpec((block_m, N_EXPERTS), lambda i: (i, 0)),
        ],
        out_shape=[
            jax.ShapeDtypeStruct((N_TOKENS, TOP_K), jnp.float32),
            jax.ShapeDtypeStruct((N_TOKENS, TOP_K), jnp.int32),
            jax.ShapeDtypeStruct((N_TOKENS, N_EXPERTS), jnp.float32),
        ],
    )(tkwt, tkit, rwt)


@jax.jit
def kernel(x, W, b):
    xf = x.reshape(N_TOKENS, D_MODEL)
    wt = W.T
    b2d = b.reshape(1, N_EXPERTS)
    logits_t = _project_t(xf, wt, b2d, block_m=1024)
    tkwt, tkit, rwt = _route_t(logits_t)
    return jnp.transpose(tkwt), jnp.transpose(tkit), jnp.transpose(rwt)
